# baseline (speedup 1.0000x reference)
import jax
import jax.numpy as jnp
from jax import lax
from jax.experimental import pallas as pl
from jax.experimental.pallas import tpu as pltpu
from jax.experimental.pallas import tpu_sc as plsc

BATCH = 16384
D = 32
LANE = 128
NC = 2
NS = 16
NW = NC * NS
BPW = BATCH // NW
NBUF = 8


def _body(uid_ref, iid_ref, utab, itab, out_hbm,
          uidx_v, iidx_v, ubufs, ibufs, out_v, usems, isems):
    wid = lax.axis_index("s") * NC + lax.axis_index("c")
    base = wid * BPW

    pltpu.sync_copy(uid_ref.at[pl.ds(base, BPW)], uidx_v.at[pl.ds(0, BPW)])
    pltpu.sync_copy(iid_ref.at[pl.ds(base, BPW)], iidx_v.at[pl.ds(0, BPW)])

    lanes = lax.iota(jnp.int32, 16)

    def ids_at(e):
        uvec = plsc.load_gather(uidx_v, [jnp.full((16,), 0, jnp.int32) + e])
        ivec = plsc.load_gather(iidx_v, [jnp.full((16,), 0, jnp.int32) + e])
        return uvec[0], ivec[0]

    def fire(e, s):
        uid, iid = ids_at(e)
        uoff = pl.multiple_of((uid >> 7) << 7, LANE)
        ioff = pl.multiple_of((iid >> 7) << 7, LANE)
        for cb in range(4):
            rsl = pl.ds(cb * 8, 8)
            pltpu.async_copy(utab.at[rsl, pl.ds(uoff, LANE)],
                             ubufs[s].at[rsl], usems[s])
            pltpu.async_copy(itab.at[rsl, pl.ds(ioff, LANE)],
                             ibufs[s].at[rsl], isems[s])

    def drain(s):
        for cb in range(4):
            rsl = pl.ds(cb * 8, 8)
            pltpu.make_async_copy(utab.at[rsl, pl.ds(0, LANE)],
                                  ubufs[s].at[rsl], usems[s]).wait()
            pltpu.make_async_copy(itab.at[rsl, pl.ds(0, LANE)],
                                  ibufs[s].at[rsl], isems[s]).wait()

    def compute(e, s):
        uid, iid = ids_at(e)
        lu = jnp.full((16,), 0, jnp.int32) + (uid & (LANE - 1))
        li = jnp.full((16,), 0, jnp.int32) + (iid & (LANE - 1))
        u_lo = plsc.load_gather(ubufs[s], [lanes, lu])
        u_hi = plsc.load_gather(ubufs[s], [lanes + 16, lu])
        i_lo = plsc.load_gather(ibufs[s], [lanes, li])
        i_hi = plsc.load_gather(ibufs[s], [lanes + 16, li])
        p = u_lo * i_lo + u_hi * i_hi
        sv = jnp.sum(p)
        plsc.store_scatter(out_v, [jnp.full((16,), 0, jnp.int32) + e],
                           jnp.full((16,), 0.0, jnp.float32) + sv,
                           mask=lanes == 0)

    for s in range(NBUF):
        fire(s, s)

    def i_body(i, _):
        for s in range(NBUF):
            e = i * NBUF + s
            drain(s)
            compute(e, s)
            fire(e + NBUF, s)
        return 0

    lax.fori_loop(0, BPW // NBUF - 1, i_body, 0)
    for s in range(NBUF):
        e = BPW - NBUF + s
        drain(s)
        compute(e, s)

    pltpu.sync_copy(out_v, out_hbm.at[pl.ds(base, BPW)])


def kernel(user_ids, item_ids, user_table, item_table):
    utT = user_table.T
    itT = item_table.T
    mesh = plsc.VectorSubcoreMesh(core_axis_name="c", subcore_axis_name="s")
    run = pl.kernel(
        _body,
        mesh=mesh,
        out_type=jax.ShapeDtypeStruct((BATCH,), jnp.float32),
        scratch_types=dict(
            uidx_v=pltpu.VMEM((BPW + 16,), jnp.int32),
            iidx_v=pltpu.VMEM((BPW + 16,), jnp.int32),
            ubufs=[pltpu.VMEM((D, LANE), jnp.float32)] * NBUF,
            ibufs=[pltpu.VMEM((D, LANE), jnp.float32)] * NBUF,
            out_v=pltpu.VMEM((BPW,), jnp.float32),
            usems=[pltpu.SemaphoreType.DMA] * NBUF,
            isems=[pltpu.SemaphoreType.DMA] * NBUF,
        ),
        compiler_params=pltpu.CompilerParams(
            needs_layout_passes=False, use_tc_tiling_on_sc=True),
    )
    return run(user_ids.astype(jnp.int32), item_ids.astype(jnp.int32),
               utT, itT)


# speedup vs baseline: 1.0098x; 1.0098x over previous
import jax
import jax.numpy as jnp
from jax import lax
from jax.experimental import pallas as pl
from jax.experimental.pallas import tpu as pltpu
from jax.experimental.pallas import tpu_sc as plsc

BATCH = 16384
D = 32
LANE = 128
NC = 2
NS = 16
NW = NC * NS
BPW = BATCH // NW
NBUF = 8


def _body(uid_ref, iid_ref, utab, itab, out_hbm,
          uidx_v, iidx_v, ubufs, ibufs, out_v, usems, isems):
    wid = lax.axis_index("s") * NC + lax.axis_index("c")
    base = wid * BPW

    pltpu.sync_copy(uid_ref.at[pl.ds(base, BPW)], uidx_v.at[pl.ds(0, BPW)])
    pltpu.sync_copy(iid_ref.at[pl.ds(base, BPW)], iidx_v.at[pl.ds(0, BPW)])

    lanes = lax.iota(jnp.int32, 16)

    def ids_at(e):
        uvec = plsc.load_gather(uidx_v, [jnp.full((16,), 0, jnp.int32) + e])
        ivec = plsc.load_gather(iidx_v, [jnp.full((16,), 0, jnp.int32) + e])
        return uvec[0], ivec[0]

    def fire(e, s):
        uid, iid = ids_at(e)
        uoff = pl.multiple_of((uid >> 7) << 7, LANE)
        ioff = pl.multiple_of((iid >> 7) << 7, LANE)
        pltpu.async_copy(utab.at[:, pl.ds(uoff, LANE)], ubufs[s], usems[s])
        pltpu.async_copy(itab.at[:, pl.ds(ioff, LANE)], ibufs[s], isems[s])

    def drain(s):
        pltpu.make_async_copy(utab.at[:, pl.ds(0, LANE)], ubufs[s],
                              usems[s]).wait()
        pltpu.make_async_copy(itab.at[:, pl.ds(0, LANE)], ibufs[s],
                              isems[s]).wait()

    def compute(e, s):
        uid, iid = ids_at(e)
        lu = jnp.full((16,), 0, jnp.int32) + (uid & (LANE - 1))
        li = jnp.full((16,), 0, jnp.int32) + (iid & (LANE - 1))
        u_lo = plsc.load_gather(ubufs[s], [lanes, lu])
        u_hi = plsc.load_gather(ubufs[s], [lanes + 16, lu])
        i_lo = plsc.load_gather(ibufs[s], [lanes, li])
        i_hi = plsc.load_gather(ibufs[s], [lanes + 16, li])
        p = u_lo * i_lo + u_hi * i_hi
        sv = jnp.sum(p)
        plsc.store_scatter(out_v, [jnp.full((16,), 0, jnp.int32) + e],
                           jnp.full((16,), 0.0, jnp.float32) + sv,
                           mask=lanes == 0)

    for s in range(NBUF):
        fire(s, s)

    def i_body(i, _):
        for s in range(NBUF):
            e = i * NBUF + s
            drain(s)
            compute(e, s)
            fire(e + NBUF, s)
        return 0

    lax.fori_loop(0, BPW // NBUF - 1, i_body, 0)
    for s in range(NBUF):
        e = BPW - NBUF + s
        drain(s)
        compute(e, s)

    pltpu.sync_copy(out_v, out_hbm.at[pl.ds(base, BPW)])


def kernel(user_ids, item_ids, user_table, item_table):
    utT = user_table.T
    itT = item_table.T
    mesh = plsc.VectorSubcoreMesh(core_axis_name="c", subcore_axis_name="s")
    run = pl.kernel(
        _body,
        mesh=mesh,
        out_type=jax.ShapeDtypeStruct((BATCH,), jnp.float32),
        scratch_types=dict(
            uidx_v=pltpu.VMEM((BPW + 16,), jnp.int32),
            iidx_v=pltpu.VMEM((BPW + 16,), jnp.int32),
            ubufs=[pltpu.VMEM((D, LANE), jnp.float32)] * NBUF,
            ibufs=[pltpu.VMEM((D, LANE), jnp.float32)] * NBUF,
            out_v=pltpu.VMEM((BPW,), jnp.float32),
            usems=[pltpu.SemaphoreType.DMA] * NBUF,
            isems=[pltpu.SemaphoreType.DMA] * NBUF,
        ),
        compiler_params=pltpu.CompilerParams(
            needs_layout_passes=False, use_tc_tiling_on_sc=True),
    )
    return run(user_ids.astype(jnp.int32), item_ids.astype(jnp.int32),
               utT, itT)
